# Initial kernel scaffold; baseline (speedup 1.0000x reference)
import functools

import jax
import jax.numpy as jnp
from jax import lax
from jax.experimental import pallas as pl
from jax.experimental.pallas import tpu as pltpu
from jax.experimental.pallas import tpu_sc as plsc

N_NODES = 10000
N_HEDGES = 10000
N_INC = 320000
NC, NS = 2, 16
NW = NC * NS
CH = 80
NCH = N_INC // NW // CH
RPT = N_HEDGES // NS
ZR = 125

_f32 = jnp.float32
_i32 = jnp.int32


def _sc_pass_body(with_counts, D, *refs):
    if with_counts:
        (tab, gi, si, out, cnt_out,
         gidx_v, sidx_v, rows_v, zrow_v, zcnt_v, ones_v, acc_sh, cnt_sh,
         sem) = refs
    else:
        (tab, gi, si, out,
         gidx_v, sidx_v, rows_v, zrow_v, acc_sh,
         sem) = refs

    c = lax.axis_index("c")
    s = lax.axis_index("s")
    wid = s * NC + c
    row0 = s * RPT

    z16 = jnp.zeros((16,), _f32)

    def zb(r, _):
        for cc in range(D // 16):
            zrow_v[r, pl.ds(cc * 16, 16)] = z16
        if with_counts:
            zcnt_v[r, pl.ds(0, 16)] = z16
        return 0

    lax.fori_loop(0, ZR, zb, 0)
    for t in range(RPT // ZR):
        pltpu.sync_copy(zrow_v, acc_sh.at[pl.ds(row0 + t * ZR, ZR)])
        if with_counts:
            pltpu.sync_copy(zcnt_v, cnt_sh.at[pl.ds(row0 + t * ZR, ZR)])

    if with_counts:
        o16 = jnp.full((16,), 1.0, _f32)

        def ob(r, _):
            ones_v[r, pl.ds(0, 16)] = o16
            return 0

        lax.fori_loop(0, CH, ob, 0)

    pltpu.sync_copy(gi.at[pl.ds(wid * NCH, NCH)], gidx_v)
    pltpu.sync_copy(si.at[pl.ds(wid * NCH, NCH)], sidx_v)

    plsc.subcore_barrier()

    def step(j, _):
        pltpu.async_copy(tab.at[gidx_v.at[j]], rows_v, sem).wait()
        pltpu.sync_copy(rows_v, acc_sh.at[sidx_v.at[j]], add=True)
        if with_counts:
            pltpu.sync_copy(ones_v, cnt_sh.at[sidx_v.at[j]], add=True)
        return 0

    lax.fori_loop(0, NCH, step, 0)

    plsc.subcore_barrier()

    pltpu.sync_copy(acc_sh.at[pl.ds(row0, RPT)],
                    out.at[c, pl.ds(row0, RPT)])
    if with_counts:
        pltpu.sync_copy(cnt_sh.at[pl.ds(row0, RPT)],
                        cnt_out.at[c, pl.ds(row0, RPT)])


@functools.lru_cache(maxsize=None)
def _sc_pass_fn(D, with_counts):
    mesh = plsc.VectorSubcoreMesh(core_axis_name="c", subcore_axis_name="s",
                                  num_cores=NC, num_subcores=NS)
    out_type = [jax.ShapeDtypeStruct((NC, N_HEDGES, D), _f32)]
    scratch = [
        pltpu.VMEM((NCH, CH), _i32),
        pltpu.VMEM((NCH, CH), _i32),
        pltpu.VMEM((CH, D), _f32),
        pltpu.VMEM((ZR, D), _f32),
    ]
    if with_counts:
        out_type.append(jax.ShapeDtypeStruct((NC, N_HEDGES, 16), _f32))
        scratch += [pltpu.VMEM((ZR, 16), _f32), pltpu.VMEM((CH, 16), _f32)]
    scratch.append(pltpu.VMEM_SHARED((N_HEDGES, D), _f32))
    if with_counts:
        scratch.append(pltpu.VMEM_SHARED((N_HEDGES, 16), _f32))
    scratch.append(pltpu.SemaphoreType.DMA)
    return pl.kernel(functools.partial(_sc_pass_body, with_counts, D),
                     out_type=tuple(out_type), mesh=mesh,
                     scratch_types=tuple(scratch))


def _sc_pass(table, gidx, sidx, with_counts=False):
    D = table.shape[1]
    return _sc_pass_fn(D, with_counts)(table, gidx, sidx)


def _mm_body(x_ref, w_ref, b_ref, o_ref):
    o_ref[...] = (jnp.dot(x_ref[...], w_ref[...],
                          preferred_element_type=_f32) + b_ref[...])


def _matmul(x, w, b, bm):
    M, K = x.shape
    N = w.shape[1]
    return pl.pallas_call(
        _mm_body,
        grid=(M // bm,),
        in_specs=[pl.BlockSpec((bm, K), lambda i: (i, 0)),
                  pl.BlockSpec((K, N), lambda i: (0, 0)),
                  pl.BlockSpec((1, N), lambda i: (0, 0))],
        out_specs=pl.BlockSpec((bm, N), lambda i: (i, 0)),
        out_shape=jax.ShapeDtypeStruct((M, N), _f32),
    )(x, w, b.reshape(1, -1))


def _merge_body(p0_ref, p1_ref, c0_ref, c1_ref, y_ref):
    cnt = (c0_ref[...] + c1_ref[...])[:, 0:1]
    y_ref[...] = (p0_ref[...] + p1_ref[...]) / jnp.maximum(cnt, 1.0)


def _merge(p0, p1, c0, c1, bm=2000):
    E, D = p0.shape
    return pl.pallas_call(
        _merge_body,
        grid=(E // bm,),
        in_specs=[pl.BlockSpec((bm, D), lambda i: (i, 0)),
                  pl.BlockSpec((bm, D), lambda i: (i, 0)),
                  pl.BlockSpec((bm, 16), lambda i: (i, 0)),
                  pl.BlockSpec((bm, 16), lambda i: (i, 0))],
        out_specs=pl.BlockSpec((bm, D), lambda i: (i, 0)),
        out_shape=jax.ShapeDtypeStruct((E, D), _f32),
    )(p0, p1, c0, c1)


def _hz_body(xw_ref, p0_ref, p1_ref, w2_ref, b2_ref, z_ref):
    h = jnp.maximum(xw_ref[...] + p0_ref[...] + p1_ref[...], 0.0)
    z_ref[...] = (jnp.dot(h, w2_ref[...],
                          preferred_element_type=_f32) + b2_ref[...])


def _hz(xw, p0, p1, w2, b2, bm=2000):
    M, K = xw.shape
    N = w2.shape[1]
    return pl.pallas_call(
        _hz_body,
        grid=(M // bm,),
        in_specs=[pl.BlockSpec((bm, K), lambda i: (i, 0)),
                  pl.BlockSpec((bm, K), lambda i: (i, 0)),
                  pl.BlockSpec((bm, K), lambda i: (i, 0)),
                  pl.BlockSpec((K, N), lambda i: (0, 0)),
                  pl.BlockSpec((1, N), lambda i: (0, 0))],
        out_specs=pl.BlockSpec((bm, N), lambda i: (i, 0)),
        out_shape=jax.ShapeDtypeStruct((M, N), _f32),
    )(xw, p0, p1, w2, b2.reshape(1, -1))


def _fin_body(z_ref, u0_ref, u1_ref, o_ref):
    o_ref[...] = z_ref[...] + u0_ref[...] + u1_ref[...]


def _fin(z, u0, u1, bm=2000):
    M, D = z.shape
    return pl.pallas_call(
        _fin_body,
        grid=(M // bm,),
        in_specs=[pl.BlockSpec((bm, D), lambda i: (i, 0))] * 3,
        out_specs=pl.BlockSpec((bm, D), lambda i: (i, 0)),
        out_shape=jax.ShapeDtypeStruct((M, D), _f32),
    )(z, u0, u1)


def kernel(x0, x1, v_idx, e_idx, W1, b1, W2, b2):
    gv = v_idx.reshape(NW * NCH, CH)
    ge = e_idx.reshape(NW * NCH, CH)

    s = _matmul(jnp.concatenate([x0, x1], 0), W1, b1, bm=2000)
    xw = jnp.concatenate([s[:N_NODES], s[N_NODES:]], 1)

    es, cnt = _sc_pass(xw, gv, ge, with_counts=True)
    y1 = _merge(es[0], es[1], cnt[0], cnt[1])

    va = _sc_pass(y1, ge, gv)
    w2p = jnp.pad(W2, ((0, 0), (0, 8)))
    b2p = jnp.pad(b2, (0, 8))
    z = _hz(xw, va[0], va[1], w2p, b2p)

    es2 = _sc_pass(z, gv, ge)
    y2 = _merge(es2[0], es2[1], cnt[0], cnt[1])
    va2 = _sc_pass(y2, ge, gv)
    out = _fin(z, va2[0], va2[1])
    return out[:, :40]


# speedup vs baseline: 5.8361x; 5.8361x over previous
import functools

import jax
import jax.numpy as jnp
from jax import lax
from jax.experimental import pallas as pl
from jax.experimental.pallas import tpu as pltpu
from jax.experimental.pallas import tpu_sc as plsc

N_NODES = 10000
N_HEDGES = 10000
N_INC = 320000
NC, NS = 2, 16
NW = NC * NS
CH = 80
NCH = N_INC // NW // CH
R_ACC = 10240
RPT = R_ACC // NS
ZR = 128

_f32 = jnp.float32
_i32 = jnp.int32


def _sc_pass_body(with_counts, D, *refs):
    if with_counts:
        (tab, gi, si, out, cnt_out,
         gidx_v, sidx_v, rows_v, zrow_v, zcnt_v, ones_v, acc_sh, cnt_sh,
         sem) = refs
    else:
        (tab, gi, si, out,
         gidx_v, sidx_v, rows_v, zrow_v, acc_sh,
         sem) = refs

    c = lax.axis_index("c")
    s = lax.axis_index("s")
    wid = s * NC + c
    row0 = s * RPT

    z16 = jnp.zeros((16,), _f32)

    def zb(r, _):
        for cc in range(D // 16):
            zrow_v[r, pl.ds(cc * 16, 16)] = z16
        if with_counts:
            zcnt_v[r, pl.ds(0, 16)] = z16
        return 0

    lax.fori_loop(0, ZR, zb, 0)
    for t in range(RPT // ZR):
        pltpu.sync_copy(zrow_v, acc_sh.at[pl.ds(row0 + t * ZR, ZR)])
        if with_counts:
            pltpu.sync_copy(zcnt_v, cnt_sh.at[pl.ds(row0 + t * ZR, ZR)])

    if with_counts:
        o16 = jnp.full((16,), 1.0, _f32)

        def ob(r, _):
            ones_v[r, pl.ds(0, 16)] = o16
            return 0

        lax.fori_loop(0, CH, ob, 0)

    pltpu.sync_copy(gi.at[wid], gidx_v)
    pltpu.sync_copy(si.at[wid], sidx_v)

    plsc.subcore_barrier()

    def step(j, _):
        pltpu.async_copy(tab.at[gidx_v.at[j]], rows_v, sem).wait()
        pltpu.sync_copy(rows_v, acc_sh.at[sidx_v.at[j]], add=True)
        if with_counts:
            pltpu.sync_copy(ones_v, cnt_sh.at[sidx_v.at[j]], add=True)
        return 0

    lax.fori_loop(0, NCH, step, 0)

    plsc.subcore_barrier()

    pltpu.sync_copy(acc_sh.at[pl.ds(row0, RPT)],
                    out.at[c, pl.ds(row0, RPT)])
    if with_counts:
        pltpu.sync_copy(cnt_sh.at[pl.ds(row0, RPT)],
                        cnt_out.at[c, pl.ds(row0, RPT)])


@functools.lru_cache(maxsize=None)
def _sc_pass_fn(D, with_counts):
    mesh = plsc.VectorSubcoreMesh(core_axis_name="c", subcore_axis_name="s",
                                  num_cores=NC, num_subcores=NS)
    out_type = [jax.ShapeDtypeStruct((NC, R_ACC, D), _f32)]
    scratch = [
        pltpu.VMEM((NCH, CH), _i32),
        pltpu.VMEM((NCH, CH), _i32),
        pltpu.VMEM((CH, D), _f32),
        pltpu.VMEM((ZR, D), _f32),
    ]
    if with_counts:
        out_type.append(jax.ShapeDtypeStruct((NC, R_ACC, 16), _f32))
        scratch += [pltpu.VMEM((ZR, 16), _f32), pltpu.VMEM((CH, 16), _f32)]
    scratch.append(pltpu.VMEM_SHARED((R_ACC, D), _f32))
    if with_counts:
        scratch.append(pltpu.VMEM_SHARED((R_ACC, 16), _f32))
    scratch.append(pltpu.SemaphoreType.DMA)
    return pl.kernel(functools.partial(_sc_pass_body, with_counts, D),
                     out_type=tuple(out_type), mesh=mesh,
                     compiler_params=pltpu.CompilerParams(
                         use_tc_tiling_on_sc=False),
                     scratch_types=tuple(scratch))


def _sc_pass(table, gidx, sidx, with_counts=False):
    D = table.shape[1]
    res = _sc_pass_fn(D, with_counts)(table, gidx, sidx)
    return res if with_counts else res[0]


def _mm_body(x_ref, w_ref, b_ref, o_ref):
    o_ref[...] = (jnp.dot(x_ref[...], w_ref[...],
                          preferred_element_type=_f32) + b_ref[...])


def _matmul(x, w, b, bm):
    M, K = x.shape
    N = w.shape[1]
    return pl.pallas_call(
        _mm_body,
        grid=(M // bm,),
        in_specs=[pl.BlockSpec((bm, K), lambda i: (i, 0)),
                  pl.BlockSpec((K, N), lambda i: (0, 0)),
                  pl.BlockSpec((1, N), lambda i: (0, 0))],
        out_specs=pl.BlockSpec((bm, N), lambda i: (i, 0)),
        out_shape=jax.ShapeDtypeStruct((M, N), _f32),
    )(x, w, b.reshape(1, -1))


def _merge_body(p0_ref, p1_ref, c0_ref, c1_ref, y_ref):
    cnt = (c0_ref[...] + c1_ref[...])[:, 0:1]
    y_ref[...] = (p0_ref[...] + p1_ref[...]) / jnp.maximum(cnt, 1.0)


def _merge(p0, p1, c0, c1, bm=2000):
    E, D = p0.shape
    return pl.pallas_call(
        _merge_body,
        grid=(E // bm,),
        in_specs=[pl.BlockSpec((bm, D), lambda i: (i, 0)),
                  pl.BlockSpec((bm, D), lambda i: (i, 0)),
                  pl.BlockSpec((bm, 16), lambda i: (i, 0)),
                  pl.BlockSpec((bm, 16), lambda i: (i, 0))],
        out_specs=pl.BlockSpec((bm, D), lambda i: (i, 0)),
        out_shape=jax.ShapeDtypeStruct((E, D), _f32),
    )(p0, p1, c0, c1)


def _hz_body(xa_ref, xb_ref, pa0_ref, pa1_ref, pb0_ref, pb1_ref,
             w2_ref, b2_ref, z_ref):
    ha = jnp.maximum(xa_ref[...] + pa0_ref[...] + pa1_ref[...], 0.0)
    hb = jnp.maximum(xb_ref[...] + pb0_ref[...] + pb1_ref[...], 0.0)
    w2 = w2_ref[...]
    z_ref[...] = (jnp.dot(ha, w2[:64], preferred_element_type=_f32)
                  + jnp.dot(hb, w2[64:], preferred_element_type=_f32)
                  + b2_ref[...])


def _hz(xa, xb, pa0, pa1, pb0, pb1, w2, b2, bm=2000):
    M, K = xa.shape
    N = w2.shape[1]
    return pl.pallas_call(
        _hz_body,
        grid=(M // bm,),
        in_specs=[pl.BlockSpec((bm, K), lambda i: (i, 0))] * 6
                 + [pl.BlockSpec((2 * K, N), lambda i: (0, 0)),
                    pl.BlockSpec((1, N), lambda i: (0, 0))],
        out_specs=pl.BlockSpec((bm, N), lambda i: (i, 0)),
        out_shape=jax.ShapeDtypeStruct((M, N), _f32),
    )(xa, xb, pa0, pa1, pb0, pb1, w2, b2.reshape(1, -1))


def _fin_body(z_ref, u0_ref, u1_ref, o_ref):
    o_ref[...] = z_ref[...] + u0_ref[...] + u1_ref[...]


def _fin(z, u0, u1, bm=2000):
    M, D = z.shape
    return pl.pallas_call(
        _fin_body,
        grid=(M // bm,),
        in_specs=[pl.BlockSpec((bm, D), lambda i: (i, 0))] * 3,
        out_specs=pl.BlockSpec((bm, D), lambda i: (i, 0)),
        out_shape=jax.ShapeDtypeStruct((M, D), _f32),
    )(z, u0, u1)


def kernel(x0, x1, v_idx, e_idx, W1, b1, W2, b2):
    gv = v_idx.reshape(NW, NCH, CH)
    ge = e_idx.reshape(NW, NCH, CH)
    N, E = N_NODES, N_HEDGES

    s = _matmul(jnp.concatenate([x0, x1], 0), W1, b1, bm=2000)
    xwa, xwb = s[:N], s[N:]

    esa, cnt = _sc_pass(xwa, gv, ge, with_counts=True)
    esb = _sc_pass(xwb, gv, ge)
    c0, c1 = cnt[0, :E], cnt[1, :E]
    y1a = _merge(esa[0, :E], esa[1, :E], c0, c1)
    y1b = _merge(esb[0, :E], esb[1, :E], c0, c1)

    vaa = _sc_pass(y1a, ge, gv)
    vab = _sc_pass(y1b, ge, gv)
    w2p = jnp.pad(W2, ((0, 0), (0, 8)))
    b2p = jnp.pad(b2, (0, 8))
    z = _hz(xwa, xwb, vaa[0, :N], vaa[1, :N], vab[0, :N], vab[1, :N],
            w2p, b2p)

    es2 = _sc_pass(z, gv, ge)
    y2 = _merge(es2[0, :E], es2[1, :E], c0, c1)
    va2 = _sc_pass(y2, ge, gv)
    out = _fin(z, va2[0, :N], va2[1, :N])
    return out[:, :40]


# speedup vs baseline: 12.3626x; 2.1183x over previous
import functools

import jax
import jax.numpy as jnp
from jax import lax
from jax.experimental import pallas as pl
from jax.experimental.pallas import tpu as pltpu
from jax.experimental.pallas import tpu_sc as plsc

N_NODES = 10000
N_HEDGES = 10000
N_INC = 320000
NC, NS = 2, 16
NW = NC * NS
CH = 128
NI_PAD = 327680
NCH = NI_PAD // NW // CH
NB = 5
R_ACC = 10240
RPT = R_ACC // NS
ZR = 128

_f32 = jnp.float32
_i32 = jnp.int32


def _sc_pass_body(with_counts, D, *refs):
    if with_counts:
        (tab, gi, si, out, cnt_out,
         gidx_v, sidx_v, rows_v, zrow_v, zcnt_v, ones_v, acc_sh, cnt_sh,
         sem) = refs
    else:
        (tab, gi, si, out,
         gidx_v, sidx_v, rows_v, zrow_v, acc_sh,
         sem) = refs

    c = lax.axis_index("c")
    s = lax.axis_index("s")
    wid = s * NC + c
    row0 = s * RPT

    z16 = jnp.zeros((16,), _f32)

    def zb(r, _):
        for cc in range(D // 16):
            zrow_v[r, pl.ds(cc * 16, 16)] = z16
        if with_counts:
            zcnt_v[r, pl.ds(0, 16)] = z16
        return 0

    lax.fori_loop(0, ZR, zb, 0)
    for t in range(RPT // ZR):
        pltpu.sync_copy(zrow_v, acc_sh.at[pl.ds(row0 + t * ZR, ZR)])
        if with_counts:
            pltpu.sync_copy(zcnt_v, cnt_sh.at[pl.ds(row0 + t * ZR, ZR)])

    if with_counts:
        o16 = jnp.full((16,), 1.0, _f32)

        def ob(r, _):
            ones_v[r, pl.ds(0, 16)] = o16
            return 0

        lax.fori_loop(0, CH, ob, 0)

    pltpu.sync_copy(gi.at[wid], gidx_v)
    pltpu.sync_copy(si.at[wid], sidx_v)

    plsc.subcore_barrier()

    gsem, ssem = sem
    for b in range(NB):
        pltpu.async_copy(tab.at[gidx_v.at[b]], rows_v.at[b], gsem)

    def super_step(it, _):
        j0 = it * NB
        for b in range(NB):
            j = j0 + b
            pltpu.make_async_copy(tab.at[gidx_v.at[j]], rows_v.at[b],
                                  gsem).wait()
            pltpu.async_copy(rows_v.at[b], acc_sh.at[sidx_v.at[j]], ssem,
                             add=True)
            if with_counts:
                pltpu.async_copy(ones_v, cnt_sh.at[sidx_v.at[j]], ssem,
                                 add=True)
        for b in range(NB):
            j = j0 + b
            pltpu.make_async_copy(rows_v.at[b], acc_sh.at[sidx_v.at[j]],
                                  ssem).wait()
            if with_counts:
                pltpu.make_async_copy(ones_v, cnt_sh.at[sidx_v.at[j]],
                                      ssem).wait()
        for b in range(NB):
            jn = j0 + NB + b

            @pl.when(jn < NCH)
            def _():
                pltpu.async_copy(tab.at[gidx_v.at[jn]], rows_v.at[b], gsem)
        return 0

    lax.fori_loop(0, NCH // NB, super_step, 0)

    plsc.subcore_barrier()

    pltpu.sync_copy(acc_sh.at[pl.ds(row0, RPT)],
                    out.at[c, pl.ds(row0, RPT)])
    if with_counts:
        pltpu.sync_copy(cnt_sh.at[pl.ds(row0, RPT)],
                        cnt_out.at[c, pl.ds(row0, RPT)])


@functools.lru_cache(maxsize=None)
def _sc_pass_fn(D, with_counts):
    mesh = plsc.VectorSubcoreMesh(core_axis_name="c", subcore_axis_name="s",
                                  num_cores=NC, num_subcores=NS)
    out_type = [jax.ShapeDtypeStruct((NC, R_ACC, D), _f32)]
    scratch = [
        pltpu.VMEM((NCH, CH), _i32),
        pltpu.VMEM((NCH, CH), _i32),
        pltpu.VMEM((NB, CH, D), _f32),
        pltpu.VMEM((ZR, D), _f32),
    ]
    if with_counts:
        out_type.append(jax.ShapeDtypeStruct((NC, R_ACC, 16), _f32))
        scratch += [pltpu.VMEM((ZR, 16), _f32), pltpu.VMEM((CH, 16), _f32)]
    scratch.append(pltpu.VMEM_SHARED((R_ACC, D), _f32))
    if with_counts:
        scratch.append(pltpu.VMEM_SHARED((R_ACC, 16), _f32))
    scratch.append((pltpu.SemaphoreType.DMA, pltpu.SemaphoreType.DMA))
    return pl.kernel(functools.partial(_sc_pass_body, with_counts, D),
                     out_type=tuple(out_type), mesh=mesh,
                     compiler_params=pltpu.CompilerParams(
                         use_tc_tiling_on_sc=False),
                     scratch_types=tuple(scratch))


def _sc_pass(table, gidx, sidx, with_counts=False):
    D = table.shape[1]
    res = _sc_pass_fn(D, with_counts)(table, gidx, sidx)
    return res if with_counts else res[0]


def _mm_body(x_ref, w_ref, b_ref, o_ref):
    o_ref[...] = (jnp.dot(x_ref[...], w_ref[...],
                          preferred_element_type=_f32) + b_ref[...])


def _matmul(x, w, b, bm):
    M, K = x.shape
    N = w.shape[1]
    return pl.pallas_call(
        _mm_body,
        grid=(M // bm,),
        in_specs=[pl.BlockSpec((bm, K), lambda i: (i, 0)),
                  pl.BlockSpec((K, N), lambda i: (0, 0)),
                  pl.BlockSpec((1, N), lambda i: (0, 0))],
        out_specs=pl.BlockSpec((bm, N), lambda i: (i, 0)),
        out_shape=jax.ShapeDtypeStruct((M, N), _f32),
    )(x, w, b.reshape(1, -1))


def _merge_body(p0_ref, p1_ref, c0_ref, c1_ref, y_ref):
    cnt = (c0_ref[...] + c1_ref[...])[:, 0:1]
    y_ref[...] = (p0_ref[...] + p1_ref[...]) / jnp.maximum(cnt, 1.0)


def _merge(p0, p1, c0, c1, bm=2000):
    E, D = p0.shape
    return pl.pallas_call(
        _merge_body,
        grid=(E // bm,),
        in_specs=[pl.BlockSpec((bm, D), lambda i: (i, 0)),
                  pl.BlockSpec((bm, D), lambda i: (i, 0)),
                  pl.BlockSpec((bm, 16), lambda i: (i, 0)),
                  pl.BlockSpec((bm, 16), lambda i: (i, 0))],
        out_specs=pl.BlockSpec((bm, D), lambda i: (i, 0)),
        out_shape=jax.ShapeDtypeStruct((E, D), _f32),
    )(p0, p1, c0, c1)


def _hz_body(xa_ref, xb_ref, pa0_ref, pa1_ref, pb0_ref, pb1_ref,
             w2_ref, b2_ref, z_ref):
    ha = jnp.maximum(xa_ref[...] + pa0_ref[...] + pa1_ref[...], 0.0)
    hb = jnp.maximum(xb_ref[...] + pb0_ref[...] + pb1_ref[...], 0.0)
    w2 = w2_ref[...]
    z_ref[...] = (jnp.dot(ha, w2[:64], preferred_element_type=_f32)
                  + jnp.dot(hb, w2[64:], preferred_element_type=_f32)
                  + b2_ref[...])


def _hz(xa, xb, pa0, pa1, pb0, pb1, w2, b2, bm=2000):
    M, K = xa.shape
    N = w2.shape[1]
    return pl.pallas_call(
        _hz_body,
        grid=(M // bm,),
        in_specs=[pl.BlockSpec((bm, K), lambda i: (i, 0))] * 6
                 + [pl.BlockSpec((2 * K, N), lambda i: (0, 0)),
                    pl.BlockSpec((1, N), lambda i: (0, 0))],
        out_specs=pl.BlockSpec((bm, N), lambda i: (i, 0)),
        out_shape=jax.ShapeDtypeStruct((M, N), _f32),
    )(xa, xb, pa0, pa1, pb0, pb1, w2, b2.reshape(1, -1))


def _fin_body(z_ref, u0_ref, u1_ref, o_ref):
    o_ref[...] = z_ref[...] + u0_ref[...] + u1_ref[...]


def _fin(z, u0, u1, bm=2000):
    M, D = z.shape
    return pl.pallas_call(
        _fin_body,
        grid=(M // bm,),
        in_specs=[pl.BlockSpec((bm, D), lambda i: (i, 0))] * 3,
        out_specs=pl.BlockSpec((bm, D), lambda i: (i, 0)),
        out_shape=jax.ShapeDtypeStruct((M, D), _f32),
    )(z, u0, u1)


def kernel(x0, x1, v_idx, e_idx, W1, b1, W2, b2):
    N, E = N_NODES, N_HEDGES
    ar = jnp.arange(NI_PAD - N_INC, dtype=jnp.int32)
    shp = (NW, NCH, CH)
    vg = jnp.concatenate([v_idx, ar % N]).reshape(shp)
    vs = jnp.concatenate([v_idx, E + ar % (R_ACC - E)]).reshape(shp)
    eg = jnp.concatenate([e_idx, ar % E]).reshape(shp)
    es_pad = jnp.concatenate([e_idx, E + ar % (R_ACC - E)]).reshape(shp)

    s = _matmul(jnp.concatenate([x0, x1], 0), W1, b1, bm=2000)
    xwa, xwb = s[:N], s[N:]

    esa, cnt = _sc_pass(xwa, vg, es_pad, with_counts=True)
    esb = _sc_pass(xwb, vg, es_pad)
    c0, c1 = cnt[0], cnt[1]
    y1a = _merge(esa[0], esa[1], c0, c1, bm=2048)
    y1b = _merge(esb[0], esb[1], c0, c1, bm=2048)

    vaa = _sc_pass(y1a, eg, vs)
    vab = _sc_pass(y1b, eg, vs)
    w2p = jnp.pad(W2, ((0, 0), (0, 8)))
    b2p = jnp.pad(b2, (0, 8))
    z = _hz(xwa, xwb, vaa[0, :N], vaa[1, :N], vab[0, :N], vab[1, :N],
            w2p, b2p)

    es2 = _sc_pass(z, vg, es_pad)
    y2 = _merge(es2[0], es2[1], c0, c1, bm=2048)
    va2 = _sc_pass(y2, eg, vs)
    out = _fin(z, va2[0, :N], va2[1, :N])
    return out[:, :40]
